# baseline (speedup 1.0000x reference)
import functools

import jax
import jax.numpy as jnp
from jax import lax
from jax.experimental import pallas as pl
from jax.experimental.pallas import tpu as pltpu
from jax.experimental.pallas import tpu_sc as plsc

N_TOTAL = 100000
D = 32
Q = 1024
K = 16
BN = 4096
NB = 25
G = 128
M = BN // G
NG = NB * G
QB = 256
NEG = float(-jnp.inf)
BIG = 2 ** 30


def _phase1_body(x_ref, mu_ref, f1_ref, f2_ref, c1_ref, c2_ref):
    blk = pl.program_id(0)
    base = blk * BN

    x = x_ref[...]
    mu = mu_ref[...]
    x_sq = jnp.sum(x * x, axis=1, keepdims=True)
    m_sq = jnp.sum(mu * mu, axis=1)[None, :]
    xm = lax.dot_general(x, mu, (((1,), (1,)), ((), ())),
                         preferred_element_type=jnp.float32)
    d = jnp.sqrt(jnp.maximum(x_sq + m_sq - 2.0 * xm, 0.0))
    iota_b = lax.broadcasted_iota(jnp.int32, (1, BN), 1)
    d = jnp.where(base + iota_b < N_TOTAL, d, NEG)

    slices = [d[:, m * G:(m + 1) * G] for m in range(M)]

    f1 = slices[0]
    for m in range(1, M):
        f1 = jnp.maximum(f1, slices[m])
    m1 = jnp.full(f1.shape, BIG, jnp.int32)
    for m in range(M - 1, -1, -1):
        m1 = jnp.where(slices[m] == f1, jnp.int32(m), m1)
    f2 = jnp.full(f1.shape, NEG, jnp.float32)
    for m in range(M):
        s = jnp.where((slices[m] == f1) & (m1 == m), NEG, slices[m])
        slices[m] = s
        f2 = jnp.maximum(f2, s)
    m2 = jnp.full(f1.shape, BIG, jnp.int32)
    for m in range(M - 1, -1, -1):
        m2 = jnp.where(slices[m] == f2, jnp.int32(m), m2)

    iota_g = lax.broadcasted_iota(jnp.int32, (1, G), 1)
    f1_ref[...] = f1
    f2_ref[...] = f2
    c1_ref[...] = base + m1 * G + iota_g
    c2_ref[...] = base + m2 * G + iota_g


@jax.jit
def _phase1(x, mu_pad):
    specs_out = [pl.BlockSpec((Q, G), lambda i: (0, i)) for _ in range(4)]
    return pl.pallas_call(
        _phase1_body,
        grid=(NB,),
        in_specs=[pl.BlockSpec((Q, D), lambda i: (0, 0)),
                  pl.BlockSpec((BN, D), lambda i: (i, 0))],
        out_specs=specs_out,
        out_shape=[jax.ShapeDtypeStruct((Q, NG), jnp.float32),
                   jax.ShapeDtypeStruct((Q, NG), jnp.float32),
                   jax.ShapeDtypeStruct((Q, NG), jnp.int32),
                   jax.ShapeDtypeStruct((Q, NG), jnp.int32)],
    )(x, mu_pad)


def _phase2_body(f1_ref, f2_ref, c1_ref, c2_ref, oidx_ref):
    f1 = f1_ref[...]
    f2 = f2_ref[...]
    c1 = c1_ref[...]
    c2 = c2_ref[...]

    idxs = []
    for _ in range(K):
        m1 = jnp.max(f1, axis=1, keepdims=True)
        m2 = jnp.max(f2, axis=1, keepdims=True)
        m = jnp.maximum(m1, m2)
        col = jnp.minimum(
            jnp.min(jnp.where(f1 == m, c1, BIG), axis=1, keepdims=True),
            jnp.min(jnp.where(f2 == m, c2, BIG), axis=1, keepdims=True))
        f1 = jnp.where((f1 == m) & (c1 == col), NEG, f1)
        f2 = jnp.where((f2 == m) & (c2 == col), NEG, f2)
        idxs.append(col)

    oidx_ref[...] = jnp.concatenate(idxs, axis=1)


@jax.jit
def _phase2(f1, f2, c1, c2):
    return pl.pallas_call(
        _phase2_body,
        grid=(Q // QB,),
        in_specs=[pl.BlockSpec((QB, NG), lambda i: (i, 0)) for _ in range(4)],
        out_specs=pl.BlockSpec((QB, K), lambda i: (i, 0)),
        out_shape=jax.ShapeDtypeStruct((Q, K), jnp.int32),
    )(f1, f2, c1, c2)


TD = 48
_NC, _NS = 2, 16
_NW = _NC * _NS
_BPW = (Q * K) // _NW

_sc_mesh = plsc.VectorSubcoreMesh(core_axis_name="c", subcore_axis_name="s")


@functools.partial(
    pl.kernel, mesh=_sc_mesh,
    compiler_params=pltpu.CompilerParams(use_tc_tiling_on_sc=False),
    out_type=jax.ShapeDtypeStruct((Q * K, TD), jnp.float32),
    scratch_types=[pltpu.VMEM((_BPW,), jnp.int32),
                   pltpu.VMEM((_BPW, TD), jnp.float32),
                   pltpu.SemaphoreType.DMA],
)
def _sc_gather(table_hbm, idx_hbm, out_hbm, idx_v, rows_v, sem):
    wid = lax.axis_index("s") * _NC + lax.axis_index("c")
    base = wid * _BPW
    pltpu.sync_copy(idx_hbm.at[pl.ds(base, _BPW)], idx_v)
    pltpu.async_copy(table_hbm.at[idx_v], rows_v, sem).wait()
    pltpu.sync_copy(rows_v, out_hbm.at[pl.ds(base, _BPW)])


def kernel(x, mu, alpha, kappa, k):
    mu_pad = jnp.concatenate(
        [mu, jnp.zeros((NB * BN - N_TOTAL, D), mu.dtype)], axis=0)
    f1, f2, c1, c2 = _phase1(x, mu_pad)
    topk_idx = _phase2(f1, f2, c1, c2)
    idx = topk_idx + (jnp.asarray(k, topk_idx.dtype) - K)
    table = jnp.concatenate(
        [mu, alpha[:, None], kappa[:, None],
         jnp.zeros((N_TOTAL, TD - D - 2), jnp.float32)], axis=1)
    g = _sc_gather(table, idx.reshape(-1))
    neighbors_mu = g[:, :D].reshape(Q, K, D)
    neighbors_alpha = g[:, D].reshape(Q, K)
    neighbors_kappa = g[:, D + 1].reshape(Q, K)
    return (neighbors_mu, neighbors_alpha, neighbors_kappa)
